# Initial kernel scaffold; baseline (speedup 1.0000x reference)
import jax
import jax.numpy as jnp
from jax.experimental import pallas as pl


def kernel(pillar_features, coords):
    raise NotImplementedError("write your pallas kernel here")


# speedup vs baseline: 2.7287x; 2.7287x over previous
import functools

import jax
import jax.numpy as jnp
from jax import lax
from jax.experimental import pallas as pl
from jax.experimental.pallas import tpu as pltpu
from jax.experimental.pallas import tpu_sc as plsc

_H, _W = 496, 432
_HW = _H * _W
_B, _P, _C = 4, 12000, 64
_PE = 12008
_NBANDS = 8
_NR = _H // _NBANDS
_CH = _NR * _W
_K = 2400
_NK = _P // _K
_VK = _K // 16
_NV = _CH // 16
_NC, _NS = 2, 16

_mesh = plsc.VectorSubcoreMesh(
    core_axis_name="c", subcore_axis_name="s", num_cores=_NC, num_subcores=_NS
)


@functools.partial(
    pl.kernel,
    out_type=jax.ShapeDtypeStruct((_B * _C * _HW,), jnp.float32),
    mesh=_mesh,
    compiler_params=pltpu.CompilerParams(needs_layout_passes=False),
    scratch_types=[
        pltpu.VMEM((_K,), jnp.int32),
        pltpu.VMEM((_K,), jnp.int32),
        pltpu.VMEM((_K,), jnp.int32),
        pltpu.VMEM((_K,), jnp.int32),
        pltpu.VMEM((_CH,), jnp.int32),
        pltpu.VMEM((_PE,), jnp.float32),
        pltpu.VMEM((_PE,), jnp.float32),
        pltpu.VMEM((_CH,), jnp.float32),
        pltpu.VMEM((_CH,), jnp.float32),
        pltpu.SemaphoreType.DMA,
        pltpu.SemaphoreType.DMA,
        pltpu.SemaphoreType.DMA,
        pltpu.SemaphoreType.DMA,
        pltpu.SemaphoreType.DMA,
        pltpu.SemaphoreType.DMA,
    ],
)
def _pseudo_image_kernel(
    y_hbm, x_hbm, ft_hbm, out_hbm,
    yb0, yb1, xb0, xb1, mapv, t0, t1, o0, o1,
    sy, sx, st0, st1, so0, so1,
):
    wid = lax.axis_index("s") * _NC + lax.axis_index("c")
    b = wid // _NBANDS
    y0 = (wid % _NBANDS) * _NR

    ybufs, xbufs = (yb0, yb1), (xb0, xb1)
    tbufs, obufs = (t0, t1), (o0, o1)
    tsems, osems = (st0, st1), (so0, so1)

    def start_in(k):
        off = pl.multiple_of(b * _P + k * _K, 8)
        cy = pltpu.async_copy(y_hbm.at[pl.ds(off, _K)], ybufs[k % 2], sy)
        cx = pltpu.async_copy(x_hbm.at[pl.ds(off, _K)], xbufs[k % 2], sx)
        return cy, cx

    in_cp = [None] * _NK
    in_cp[0] = start_in(0)

    empty = jnp.full((16,), _P, dtype=jnp.int32)

    def init_body(v, _):
        mapv[pl.ds(v * 16, 16)] = empty
        return 0

    lax.fori_loop(0, _NV, init_body, 0)

    i16 = lax.iota(jnp.int32, 16)

    for k in range(_NK):
        if k + 1 < _NK:
            in_cp[k + 1] = start_in(k + 1)
        in_cp[k][0].wait()
        in_cp[k][1].wait()
        yb, xb = ybufs[k % 2], xbufs[k % 2]
        base = k * _K

        def chunk_body(v, _, yb=yb, xb=xb, base=base):
            yv = yb[pl.ds(v * 16, 16)]
            xv = xb[pl.ds(v * 16, 16)]
            valid = (xv >= 0) & (xv < _W) & (yv >= y0) & (yv < y0 + _NR)
            flat = (yv - y0) * _W + xv
            p = base + v * 16 + i16
            plsc.store_scatter(mapv, [flat], p, mask=valid)
            for _r in range(2):
                rb = plsc.load_gather(mapv, [flat], mask=valid)
                m2 = valid & (p > rb)
                plsc.store_scatter(mapv, [flat], p, mask=m2)
            return 0

        lax.fori_loop(0, _VK, chunk_body, 0)

    t_cp = [None, None]
    o_cp = [None, None]

    def tab_off(c):
        return pl.multiple_of((b * _C + c) * _PE, 8)

    t_cp[0] = pltpu.async_copy(ft_hbm.at[pl.ds(tab_off(0), _PE)], t0, st0)

    for c in range(_C):
        par = c % 2
        tb, ob = tbufs[par], obufs[par]
        if c + 1 < _C:
            t_cp[1 - par] = pltpu.async_copy(
                ft_hbm.at[pl.ds(tab_off(c + 1), _PE)], tbufs[1 - par], tsems[1 - par]
            )
        t_cp[par].wait()
        if o_cp[par] is not None:
            o_cp[par].wait()

        def gather_body(v, _, tb=tb, ob=ob):
            idx = mapv[pl.ds(v * 16, 16)]
            ob[pl.ds(v * 16, 16)] = plsc.load_gather(tb, [idx])
            return 0

        lax.fori_loop(0, _NV, gather_body, 0)
        out_off = pl.multiple_of((b * _C + c) * _HW + y0 * _W, 8)
        o_cp[par] = pltpu.async_copy(
            ob, out_hbm.at[pl.ds(out_off, _CH)], osems[par]
        )

    o_cp[0].wait()
    o_cp[1].wait()


def kernel(pillar_features, coords):
    y = coords[:, :, 1].astype(jnp.int32).reshape(-1)
    x = coords[:, :, 2].astype(jnp.int32).reshape(-1)
    ft = jnp.transpose(pillar_features, (0, 2, 1)).astype(jnp.float32)
    ft = jnp.pad(ft, ((0, 0), (0, 0), (0, _PE - _P))).reshape(-1)
    out = _pseudo_image_kernel(y, x, ft)
    return out.reshape(_B, _C, _H, _W)


# speedup vs baseline: 3.7294x; 1.3667x over previous
import functools

import jax
import jax.numpy as jnp
from jax import lax
from jax.experimental import pallas as pl
from jax.experimental.pallas import tpu as pltpu
from jax.experimental.pallas import tpu_sc as plsc

_H, _W = 496, 432
_HW = _H * _W
_B, _P, _C = 4, 12000, 64
_PE = 12008
_NBANDS = 8
_NR = _H // _NBANDS
_CH = _NR * _W
_K = 2400
_NK = _P // _K
_VK = _K // 16
_NV = _CH // 16
_NC, _NS = 2, 16

_mesh = plsc.VectorSubcoreMesh(
    core_axis_name="c", subcore_axis_name="s", num_cores=_NC, num_subcores=_NS
)


@functools.partial(
    pl.kernel,
    out_type=jax.ShapeDtypeStruct((_B * _C * _HW,), jnp.float32),
    mesh=_mesh,
    compiler_params=pltpu.CompilerParams(needs_layout_passes=False),
    scratch_types=[
        pltpu.VMEM((_K,), jnp.int32),
        pltpu.VMEM((_K,), jnp.int32),
        pltpu.VMEM((_K,), jnp.int32),
        pltpu.VMEM((_K,), jnp.int32),
        pltpu.VMEM((_CH,), jnp.int32),
        pltpu.VMEM((_PE,), jnp.float32),
        pltpu.VMEM((_PE,), jnp.float32),
        pltpu.VMEM((_CH,), jnp.float32),
        pltpu.VMEM((_CH,), jnp.float32),
        pltpu.SemaphoreType.DMA,
        pltpu.SemaphoreType.DMA,
        pltpu.SemaphoreType.DMA,
        pltpu.SemaphoreType.DMA,
        pltpu.SemaphoreType.DMA,
        pltpu.SemaphoreType.DMA,
    ],
)
def _pseudo_image_kernel(
    y_hbm, x_hbm, ft_hbm, out_hbm,
    yb0, yb1, xb0, xb1, mapv, t0, t1, o0, o1,
    sy, sx, st0, st1, so0, so1,
):
    wid = lax.axis_index("s") * _NC + lax.axis_index("c")
    b = wid // _NBANDS
    y0 = (wid % _NBANDS) * _NR

    ybufs, xbufs = (yb0, yb1), (xb0, xb1)
    tbufs, obufs = (t0, t1), (o0, o1)
    tsems, osems = (st0, st1), (so0, so1)

    def start_in(k):
        off = pl.multiple_of(b * _P + k * _K, 8)
        cy = pltpu.async_copy(y_hbm.at[pl.ds(off, _K)], ybufs[k % 2], sy)
        cx = pltpu.async_copy(x_hbm.at[pl.ds(off, _K)], xbufs[k % 2], sx)
        return cy, cx

    in_cp = [None] * _NK
    in_cp[0] = start_in(0)

    empty = jnp.full((16,), _P, dtype=jnp.int32)

    @plsc.parallel_loop(0, _NV, unroll=6)
    def _init_body(v):
        mapv[pl.ds(v * 16, 16)] = empty

    i16 = lax.iota(jnp.int32, 16)

    for k in range(_NK):
        if k + 1 < _NK:
            in_cp[k + 1] = start_in(k + 1)
        in_cp[k][0].wait()
        in_cp[k][1].wait()
        yb, xb = ybufs[k % 2], xbufs[k % 2]
        base = k * _K

        def chunk_body(v, _, yb=yb, xb=xb, base=base):
            yv = yb[pl.ds(v * 16, 16)]
            xv = xb[pl.ds(v * 16, 16)]
            valid = (xv >= 0) & (xv < _W) & (yv >= y0) & (yv < y0 + _NR)
            flat = (yv - y0) * _W + xv
            p = base + v * 16 + i16
            plsc.store_scatter(mapv, [flat], p, mask=valid)
            for _r in range(2):
                rb = plsc.load_gather(mapv, [flat], mask=valid)
                m2 = valid & (p > rb)
                plsc.store_scatter(mapv, [flat], p, mask=m2)
            return 0

        lax.fori_loop(0, _VK, chunk_body, 0)

    t_cp = [None, None]
    o_cp = [None, None]

    def tab_off(c):
        return pl.multiple_of((b * _C + c) * _PE, 8)

    t_cp[0] = pltpu.async_copy(ft_hbm.at[pl.ds(tab_off(0), _PE)], t0, st0)

    for c in range(_C):
        par = c % 2
        tb, ob = tbufs[par], obufs[par]
        if c + 1 < _C:
            t_cp[1 - par] = pltpu.async_copy(
                ft_hbm.at[pl.ds(tab_off(c + 1), _PE)], tbufs[1 - par], tsems[1 - par]
            )
        t_cp[par].wait()
        if o_cp[par] is not None:
            o_cp[par].wait()

        @plsc.parallel_loop(0, _NV, unroll=6)
        def _gather_body(v):
            idx = mapv[pl.ds(v * 16, 16)]
            ob[pl.ds(v * 16, 16)] = plsc.load_gather(tb, [idx])
        out_off = pl.multiple_of((b * _C + c) * _HW + y0 * _W, 8)
        o_cp[par] = pltpu.async_copy(
            ob, out_hbm.at[pl.ds(out_off, _CH)], osems[par]
        )

    o_cp[0].wait()
    o_cp[1].wait()


def kernel(pillar_features, coords):
    y = coords[:, :, 1].astype(jnp.int32).reshape(-1)
    x = coords[:, :, 2].astype(jnp.int32).reshape(-1)
    ft = jnp.transpose(pillar_features, (0, 2, 1)).astype(jnp.float32)
    ft = jnp.pad(ft, ((0, 0), (0, 0), (0, _PE - _P))).reshape(-1)
    out = _pseudo_image_kernel(y, x, ft)
    return out.reshape(_B, _C, _H, _W)


# speedup vs baseline: 3.8051x; 1.0203x over previous
import functools

import jax
import jax.numpy as jnp
from jax import lax
from jax.experimental import pallas as pl
from jax.experimental.pallas import tpu as pltpu
from jax.experimental.pallas import tpu_sc as plsc

_H, _W = 496, 432
_HW = _H * _W
_B, _P, _C = 4, 12000, 64
_PE = 12008
_NBANDS = 8
_NR = _H // _NBANDS
_CH = _NR * _W
_K = 2400
_NK = _P // _K
_VK = _K // 16
_NV = _CH // 16
_NC, _NS = 2, 16

_mesh = plsc.VectorSubcoreMesh(
    core_axis_name="c", subcore_axis_name="s", num_cores=_NC, num_subcores=_NS
)


@functools.partial(
    pl.kernel,
    out_type=jax.ShapeDtypeStruct((_B * _C * _HW,), jnp.float32),
    mesh=_mesh,
    compiler_params=pltpu.CompilerParams(needs_layout_passes=False),
    scratch_types=[
        pltpu.VMEM((_K,), jnp.int32),
        pltpu.VMEM((_K,), jnp.int32),
        pltpu.VMEM((_K,), jnp.int32),
        pltpu.VMEM((_K,), jnp.int32),
        pltpu.VMEM((_CH,), jnp.int32),
        pltpu.VMEM((_P + 16,), jnp.int32),
        pltpu.VMEM((_PE,), jnp.float32),
        pltpu.VMEM((_PE,), jnp.float32),
        pltpu.VMEM((_CH + 16,), jnp.float32),
        pltpu.VMEM((_CH + 16,), jnp.float32),
        pltpu.SemaphoreType.DMA,
        pltpu.SemaphoreType.DMA,
        pltpu.SemaphoreType.DMA,
        pltpu.SemaphoreType.DMA,
        pltpu.SemaphoreType.DMA,
        pltpu.SemaphoreType.DMA,
    ],
)
def _pseudo_image_kernel(
    y_hbm, x_hbm, ft_hbm, out_hbm,
    yb0, yb1, xb0, xb1, mapv, listv, t0, t1, o0, o1,
    sy, sx, st0, st1, so0, so1,
):
    wid = lax.axis_index("s") * _NC + lax.axis_index("c")
    b = wid // _NBANDS
    y0 = (wid % _NBANDS) * _NR

    ybufs, xbufs = (yb0, yb1), (xb0, xb1)
    tbufs, obufs = (t0, t1), (o0, o1)
    tsems, osems = (st0, st1), (so0, so1)

    def start_in(k):
        off = pl.multiple_of(b * _P + k * _K, 8)
        cy = pltpu.async_copy(y_hbm.at[pl.ds(off, _K)], ybufs[k % 2], sy)
        cx = pltpu.async_copy(x_hbm.at[pl.ds(off, _K)], xbufs[k % 2], sx)
        return cy, cx

    in_cp = [None] * _NK
    in_cp[0] = start_in(0)

    empty = jnp.full((16,), _P, dtype=jnp.int32)

    @plsc.parallel_loop(0, _NV, unroll=6)
    def _init_body(v):
        mapv[pl.ds(v * 16, 16)] = empty

    zero16 = jnp.zeros((16,), jnp.float32)

    @plsc.parallel_loop(0, (_CH + 16) // 16, unroll=6)
    def _z0(v):
        o0[pl.ds(v * 16, 16)] = zero16

    @plsc.parallel_loop(0, (_CH + 16) // 16, unroll=6)
    def _z1(v):
        o1[pl.ds(v * 16, 16)] = zero16

    i16 = lax.iota(jnp.int32, 16)

    for k in range(_NK):
        if k + 1 < _NK:
            in_cp[k + 1] = start_in(k + 1)
        in_cp[k][0].wait()
        in_cp[k][1].wait()
        yb, xb = ybufs[k % 2], xbufs[k % 2]
        base = k * _K

        def chunk_body(v, _, yb=yb, xb=xb, base=base):
            yv = yb[pl.ds(v * 16, 16)]
            xv = xb[pl.ds(v * 16, 16)]
            valid = (xv >= 0) & (xv < _W) & (yv >= y0) & (yv < y0 + _NR)
            flat = (yv - y0) * _W + xv
            p = base + v * 16 + i16
            plsc.store_scatter(mapv, [flat], p, mask=valid)
            for _r in range(2):
                rb = plsc.load_gather(mapv, [flat], mask=valid)
                m2 = valid & (p > rb)
                plsc.store_scatter(mapv, [flat], p, mask=m2)
            return 0

        lax.fori_loop(0, _VK, chunk_body, 0)

    @plsc.parallel_loop(0, _NV, unroll=2, carry=jnp.int32(0))
    def cnt(v, n):
        m = mapv[pl.ds(v * 16, 16)]
        keep = m != _P
        w = ((v * 16 + i16) << 14) | m
        plsc.store_compressed(listv.at[pl.ds(n, 16)], w, mask=keep)
        return n + jnp.sum(keep.astype(jnp.int32))

    listv[pl.ds(cnt, 16)] = jnp.full((16,), (_CH << 14) | _P, dtype=jnp.int32)
    ngroups = (cnt + 15) // 16

    t_cp = [None, None]
    o_cp = [None, None]

    def tab_off(c):
        return pl.multiple_of((b * _C + c) * _PE, 8)

    t_cp[0] = pltpu.async_copy(ft_hbm.at[pl.ds(tab_off(0), _PE)], t0, st0)

    for c in range(_C):
        par = c % 2
        tb, ob = tbufs[par], obufs[par]
        if c + 1 < _C:
            t_cp[1 - par] = pltpu.async_copy(
                ft_hbm.at[pl.ds(tab_off(c + 1), _PE)], tbufs[1 - par], tsems[1 - par]
            )
        t_cp[par].wait()
        if o_cp[par] is not None:
            o_cp[par].wait()

        @plsc.parallel_loop(0, ngroups, unroll=4)
        def _val_body(g):
            w = listv[pl.ds(g * 16, 16)]
            cell = lax.shift_right_logical(w, 14)
            p = w & 0x3FFF
            plsc.store_scatter(ob, [cell], plsc.load_gather(tb, [p]))

        out_off = pl.multiple_of((b * _C + c) * _HW + y0 * _W, 8)
        o_cp[par] = pltpu.async_copy(
            ob.at[pl.ds(0, _CH)], out_hbm.at[pl.ds(out_off, _CH)], osems[par]
        )

    o_cp[0].wait()
    o_cp[1].wait()


def kernel(pillar_features, coords):
    y = coords[:, :, 1].astype(jnp.int32).reshape(-1)
    x = coords[:, :, 2].astype(jnp.int32).reshape(-1)
    ft = jnp.transpose(pillar_features, (0, 2, 1)).astype(jnp.float32)
    ft = jnp.pad(ft, ((0, 0), (0, 0), (0, _PE - _P))).reshape(-1)
    out = _pseudo_image_kernel(y, x, ft)
    return out.reshape(_B, _C, _H, _W)
